# Initial kernel scaffold; baseline (speedup 1.0000x reference)
import jax
import jax.numpy as jnp
from jax.experimental import pallas as pl


def kernel(x, W_enc, b_enc, gamma, beta, embeddings, W_dec, b_dec):
    raise NotImplementedError("write your pallas kernel here")


# speedup vs baseline: 1.5819x; 1.5819x over previous
import functools

import jax
import jax.numpy as jnp
from jax.experimental import pallas as pl
from jax.experimental.pallas import tpu as pltpu

_B = 16384
_D_IN = 768
_D_CODE = 64
_K = 512
_BLK = 1024
_NB = _B // _BLK


def _body(x_ref, we_ref, be_ref, g_ref, bt_ref, emb_ref, wd_ref, bd_ref,
          xr_ref, idx_ref, ze_ref, zq_ref, loss_ref, edb_ref):
    i = pl.program_id(0)

    @pl.when(i == 0)
    def _():
        edb_ref[...] = (
            jnp.dot(emb_ref[...], wd_ref[...],
                    preferred_element_type=jnp.float32) + bd_ref[...])
        loss_ref[...] = jnp.zeros((1, 1), jnp.float32)

    x = x_ref[...]
    h = jnp.dot(x, we_ref[...],
                preferred_element_type=jnp.float32) + be_ref[...]
    mu = jnp.mean(h, axis=1, keepdims=True)
    hc = h - mu
    var = jnp.mean(hc * hc, axis=1, keepdims=True)
    z_e = hc / jnp.sqrt(var + 1e-5) * g_ref[...] + bt_ref[...]

    emb = emb_ref[...]
    d = (jnp.sum(z_e * z_e, axis=1, keepdims=True)
         - 2.0 * jax.lax.dot_general(
             z_e, emb, (((1,), (1,)), ((), ())),
             preferred_element_type=jnp.float32)
         + jnp.sum(emb * emb, axis=1)[None, :])

    iota = jax.lax.broadcasted_iota(jnp.int32, (_BLK, _K), 1)
    dmin = jnp.min(d, axis=1, keepdims=True)
    idx = jnp.min(jnp.where(d == dmin, iota, _K), axis=1)
    idx_ref[...] = idx.reshape(_BLK, 1)

    onehot = (iota == idx[:, None]).astype(jnp.bfloat16)
    z_q = jnp.dot(onehot, emb.astype(jnp.bfloat16),
                  preferred_element_type=jnp.float32)
    ze_ref[...] = z_e
    zq_ref[...] = z_q

    diff = z_e - z_q
    loss_ref[...] += jnp.sum(diff * diff).reshape(1, 1)

    xr_ref[...] = jnp.dot(onehot, edb_ref[...].astype(jnp.bfloat16),
                          preferred_element_type=jnp.float32)


@functools.partial(jax.jit, static_argnames=())
def kernel(x, W_enc, b_enc, gamma, beta, embeddings, W_dec, b_dec):
    be2 = b_enc.reshape(1, _D_CODE)
    g2 = gamma.reshape(1, _D_CODE)
    bt2 = beta.reshape(1, _D_CODE)
    bd2 = b_dec.reshape(1, _D_IN)

    xr, idx2, ze, zq, loss_sum = pl.pallas_call(
        _body,
        grid=(_NB,),
        in_specs=[
            pl.BlockSpec((_BLK, _D_IN), lambda i: (i, 0)),
            pl.BlockSpec((_D_IN, _D_CODE), lambda i: (0, 0)),
            pl.BlockSpec((1, _D_CODE), lambda i: (0, 0)),
            pl.BlockSpec((1, _D_CODE), lambda i: (0, 0)),
            pl.BlockSpec((1, _D_CODE), lambda i: (0, 0)),
            pl.BlockSpec((_K, _D_CODE), lambda i: (0, 0)),
            pl.BlockSpec((_D_CODE, _D_IN), lambda i: (0, 0)),
            pl.BlockSpec((1, _D_IN), lambda i: (0, 0)),
        ],
        out_specs=[
            pl.BlockSpec((_BLK, _D_IN), lambda i: (i, 0)),
            pl.BlockSpec((_BLK, 1), lambda i: (i, 0)),
            pl.BlockSpec((_BLK, _D_CODE), lambda i: (i, 0)),
            pl.BlockSpec((_BLK, _D_CODE), lambda i: (i, 0)),
            pl.BlockSpec((1, 1), lambda i: (0, 0)),
        ],
        out_shape=[
            jax.ShapeDtypeStruct((_B, _D_IN), jnp.float32),
            jax.ShapeDtypeStruct((_B, 1), jnp.int32),
            jax.ShapeDtypeStruct((_B, _D_CODE), jnp.float32),
            jax.ShapeDtypeStruct((_B, _D_CODE), jnp.float32),
            jax.ShapeDtypeStruct((1, 1), jnp.float32),
        ],
        scratch_shapes=[pltpu.VMEM((_K, _D_IN), jnp.float32)],
    )(x, W_enc, be2, g2, bt2, embeddings, W_dec, bd2)

    commitment_loss = loss_sum[0, 0] / (_B * _D_CODE)
    return (xr, commitment_loss, idx2.reshape(_B), ze, zq)
